# baseline (speedup 1.0000x reference)
import functools

import jax
import jax.numpy as jnp
from jax import lax
from jax.experimental import pallas as pl
from jax.experimental.pallas import tpu as pltpu
from jax.experimental.pallas import tpu_sc as plsc

DIM = 1024
T_LEN = 16
H_LEN = 32
W_LEN = 32
NUM_ROWS = T_LEN * H_LEN * W_LEN
LANES = 16
NVEC = DIM // LANES
NW = 32
ROWS_PER_W = NUM_ROWS // NW
HI_PER_W = 16
CHUNK_ROWS = W_LEN

_mesh = plsc.VectorSubcoreMesh(core_axis_name="c", subcore_axis_name="s")


@functools.partial(
    pl.kernel,
    mesh=_mesh,
    out_type=jax.ShapeDtypeStruct((NUM_ROWS, DIM), jnp.float32),
    scratch_types=[
        pltpu.VMEM((1, DIM), jnp.float32),
        pltpu.VMEM((HI_PER_W, DIM), jnp.float32),
        pltpu.VMEM((W_LEN, DIM), jnp.float32),
        pltpu.VMEM((LANES,), jnp.float32),
        pltpu.VMEM((LANES,), jnp.float32),
        pltpu.VMEM((LANES,), jnp.float32),
        pltpu.VMEM((2, CHUNK_ROWS, DIM), jnp.float32),
        pltpu.SemaphoreType.DMA,
        pltpu.SemaphoreType.DMA,
    ],
)
def _pos_embed_sc(t_hbm, h_hbm, w_hbm, st_hbm, sh_hbm, sw_hbm, out_hbm,
                  trow_v, th_v, w_v, st_v, sh_v, sw_v, obuf, sem0, sem1):
    wid = lax.axis_index("s") * 2 + lax.axis_index("c")
    ti = wid // 2
    half = wid % 2

    cps = [
        pltpu.async_copy(t_hbm.at[pl.ds(ti, 1)], trow_v, sem0),
        pltpu.async_copy(h_hbm.at[pl.ds(half * HI_PER_W, HI_PER_W)], th_v, sem0),
        pltpu.async_copy(st_hbm, st_v, sem0),
        pltpu.async_copy(sh_hbm, sh_v, sem0),
        pltpu.async_copy(sw_hbm, sw_v, sem0),
    ]
    w_cp = pltpu.async_copy(w_hbm, w_v, sem1)
    for c in cps:
        c.wait()

    stv = st_v[...]
    shv = sh_v[...]
    swv = sw_v[...]

    @plsc.parallel_loop(0, NVEC)
    def _prescale(j):
        sl = pl.ds(j * LANES, LANES)
        tj = trow_v[0, sl] * stv
        for hi in range(HI_PER_W):
            th_v[hi, sl] = th_v[hi, sl] * shv + tj

    w_cp.wait()

    base = wid * ROWS_PER_W
    sems = (sem0, sem1)
    pending = [None, None]
    for hi in range(HI_PER_W):
        b = hi % 2
        if pending[b] is not None:
            pending[b].wait()

        @plsc.parallel_loop(0, NVEC)
        def _chunk(j, _hi=hi, _b=b):
            sl = pl.ds(j * LANES, LANES)
            thj = th_v[_hi, sl]
            for wi in range(W_LEN):
                obuf[_b, wi, sl] = thj + w_v[wi, sl] * swv

        pending[b] = pltpu.async_copy(
            obuf.at[b],
            out_hbm.at[pl.ds(base + hi * CHUNK_ROWS, CHUNK_ROWS)],
            sems[b],
        )
    pending[0].wait()
    pending[1].wait()


def kernel(t, h, w, temporal_embed, height_embed, width_embed,
           scale_t, scale_h, scale_w):
    t0 = (t - T_LEN).astype(jnp.int32)
    h0 = (h - H_LEN).astype(jnp.int32)
    w0 = (w - W_LEN).astype(jnp.int32)
    t_tab = lax.dynamic_slice(temporal_embed, (t0, 0), (T_LEN, DIM))
    h_tab = lax.dynamic_slice(height_embed, (h0, 0), (H_LEN, DIM))
    w_tab = lax.dynamic_slice(width_embed, (w0, 0), (W_LEN, DIM))
    st = jnp.broadcast_to(scale_t.astype(jnp.float32), (LANES,))
    sh = jnp.broadcast_to(scale_h.astype(jnp.float32), (LANES,))
    sw = jnp.broadcast_to(scale_w.astype(jnp.float32), (LANES,))
    out = _pos_embed_sc(t_tab, h_tab, w_tab, st, sh, sw)
    return out.reshape(1, NUM_ROWS, DIM)


# speedup vs baseline: 1.0172x; 1.0172x over previous
import functools

import jax
import jax.numpy as jnp
from jax import lax
from jax.experimental import pallas as pl
from jax.experimental.pallas import tpu as pltpu
from jax.experimental.pallas import tpu_sc as plsc

DIM = 1024
T_LEN = 16
H_LEN = 32
W_LEN = 32
NUM_ROWS = T_LEN * H_LEN * W_LEN
LANES = 16
NVEC = DIM // LANES
NW = 32
ROWS_PER_W = NUM_ROWS // NW
HI_PER_W = 16
CHUNK_ROWS = W_LEN

_mesh = plsc.VectorSubcoreMesh(core_axis_name="c", subcore_axis_name="s")


@functools.partial(
    pl.kernel,
    mesh=_mesh,
    out_type=jax.ShapeDtypeStruct((NUM_ROWS, DIM), jnp.float32),
    scratch_types=[
        pltpu.VMEM((1, DIM), jnp.float32),
        pltpu.VMEM((HI_PER_W, DIM), jnp.float32),
        pltpu.VMEM((W_LEN, DIM), jnp.float32),
        pltpu.VMEM((LANES,), jnp.float32),
        pltpu.VMEM((LANES,), jnp.float32),
        pltpu.VMEM((LANES,), jnp.float32),
        pltpu.VMEM((2, CHUNK_ROWS, DIM), jnp.float32),
        pltpu.SemaphoreType.DMA,
        pltpu.SemaphoreType.DMA,
    ],
)
def _pos_embed_sc(t_hbm, h_hbm, w_hbm, st_hbm, sh_hbm, sw_hbm, out_hbm,
                  trow_v, th_v, w_v, st_v, sh_v, sw_v, obuf, sem0, sem1):
    wid = lax.axis_index("s") * 2 + lax.axis_index("c")
    ti = wid // 2
    half = wid % 2

    cps = [
        pltpu.async_copy(t_hbm.at[pl.ds(ti, 1)], trow_v, sem0),
        pltpu.async_copy(h_hbm.at[pl.ds(half * HI_PER_W, HI_PER_W)], th_v, sem0),
        pltpu.async_copy(st_hbm, st_v, sem0),
        pltpu.async_copy(sh_hbm, sh_v, sem0),
        pltpu.async_copy(sw_hbm, sw_v, sem0),
    ]
    w_cp = pltpu.async_copy(w_hbm, w_v, sem1)
    for c in cps:
        c.wait()

    stv = st_v[...]
    shv = sh_v[...]
    swv = sw_v[...]

    @plsc.parallel_loop(0, NVEC)
    def _prescale(j):
        sl = pl.ds(j * LANES, LANES)
        tj = trow_v[0, sl] * stv
        for hi in range(HI_PER_W):
            th_v[hi, sl] = th_v[hi, sl] * shv + tj

    w_cp.wait()

    base = wid * ROWS_PER_W
    sems = (sem0, sem1)
    pending = [None, None]
    for hi in range(HI_PER_W):
        b = hi % 2
        if pending[b] is not None:
            pending[b].wait()

        @plsc.parallel_loop(0, NVEC, unroll=2)
        def _chunk(j, _hi=hi, _b=b):
            sl = pl.ds(j * LANES, LANES)
            thj = th_v[_hi, sl]

            @plsc.parallel_loop(0, W_LEN, unroll=8)
            def _inner(wi, _sl=sl, _thj=thj):
                obuf[_b, wi, _sl] = _thj + w_v[wi, _sl] * swv

        pending[b] = pltpu.async_copy(
            obuf.at[b],
            out_hbm.at[pl.ds(base + hi * CHUNK_ROWS, CHUNK_ROWS)],
            sems[b],
        )
    pending[0].wait()
    pending[1].wait()


def kernel(t, h, w, temporal_embed, height_embed, width_embed,
           scale_t, scale_h, scale_w):
    t0 = (t - T_LEN).astype(jnp.int32)
    h0 = (h - H_LEN).astype(jnp.int32)
    w0 = (w - W_LEN).astype(jnp.int32)
    t_tab = lax.dynamic_slice(temporal_embed, (t0, 0), (T_LEN, DIM))
    h_tab = lax.dynamic_slice(height_embed, (h0, 0), (H_LEN, DIM))
    w_tab = lax.dynamic_slice(width_embed, (w0, 0), (W_LEN, DIM))
    st = jnp.broadcast_to(scale_t.astype(jnp.float32), (LANES,))
    sh = jnp.broadcast_to(scale_h.astype(jnp.float32), (LANES,))
    sw = jnp.broadcast_to(scale_w.astype(jnp.float32), (LANES,))
    out = _pos_embed_sc(t_tab, h_tab, w_tab, st, sh, sw)
    return out.reshape(1, NUM_ROWS, DIM)
